# baseline (speedup 1.0000x reference)
import functools

import jax
import jax.numpy as jnp
from jax import lax
from jax.experimental import pallas as pl
from jax.experimental.pallas import tpu as pltpu
from jax.experimental.pallas import tpu_sc as plsc

N_NODES = 10000
N_EDGES = 320000
D = 128

NC = 2
NS = 16
NW = NC * NS

EDGE_BATCH = 80
K_GROUP = 42
N_GROUPS = 3
N_BATCHES = K_GROUP * N_GROUPS
EDGES_PER_TILE = N_BATCHES * EDGE_BATCH
E_PAD = EDGES_PER_TILE * NW
N_PAD = 10240
ROWS_PER_TILE = N_PAD // NS


def _sc_aggregate(x, row, col, zeros_blk):
    mesh = plsc.VectorSubcoreMesh(core_axis_name="c", subcore_axis_name="s")

    @functools.partial(
        pl.kernel,
        mesh=mesh,
        out_type=jax.ShapeDtypeStruct((NC, N_PAD, D), jnp.float32),
        scratch_types=[
            pltpu.VMEM((K_GROUP, EDGE_BATCH), jnp.int32),
            pltpu.VMEM((K_GROUP, EDGE_BATCH), jnp.int32),
            pltpu.VMEM((EDGE_BATCH, D), jnp.float32),
            pltpu.VMEM((EDGE_BATCH, D), jnp.float32),
            pltpu.VMEM_SHARED((N_PAD, D), jnp.float32),
            pltpu.SemaphoreType.DMA,
            pltpu.SemaphoreType.DMA,
            pltpu.SemaphoreType.DMA,
            pltpu.SemaphoreType.DMA,
        ],
    )
    def k(x_hbm, row_hbm, col_hbm, zeros_hbm, out_hbm,
          row_v, col_v, rows0, rows1, agg, sg0, sg1, ss0, ss1):
        c = lax.axis_index("c")
        s = lax.axis_index("s")
        wid = s * NC + c
        rows = (rows0, rows1)
        sg = (sg0, sg1)
        ss = (ss0, ss1)

        pltpu.sync_copy(zeros_hbm, agg.at[pl.ds(s * ROWS_PER_TILE, ROWS_PER_TILE)])
        plsc.subcore_barrier()

        def group(g, carry):
            pltpu.sync_copy(row_hbm.at[wid, g], row_v)
            pltpu.sync_copy(col_hbm.at[wid, g], col_v)
            @pl.when(g > 0)
            def _():
                pltpu.make_async_copy(
                    rows[0], agg.at[col_v.at[0]], ss[0]).wait()
            pltpu.async_copy(x_hbm.at[row_v.at[0]], rows[0], sg[0])

            for j in range(K_GROUP):
                p = j % 2
                q = (j + 1) % 2
                pltpu.make_async_copy(
                    x_hbm.at[row_v.at[j]], rows[p], sg[p]).wait()
                pltpu.async_copy(rows[p], agg.at[col_v.at[j]], ss[p], add=True)
                if j + 1 < K_GROUP:
                    if j == 0:
                        @pl.when(g > 0)
                        def _():
                            pltpu.make_async_copy(
                                rows[1], agg.at[col_v.at[j]], ss[1]).wait()
                    else:
                        pltpu.make_async_copy(
                            rows[q], agg.at[col_v.at[j]], ss[q]).wait()
                    pltpu.async_copy(x_hbm.at[row_v.at[j + 1]], rows[q], sg[q])
            return carry

        lax.fori_loop(0, N_GROUPS, group, 0)
        pltpu.make_async_copy(rows[0], agg.at[col_v.at[0]], ss[0]).wait()
        pltpu.make_async_copy(rows[1], agg.at[col_v.at[1]], ss[1]).wait()
        plsc.subcore_barrier()

        r0 = s * ROWS_PER_TILE
        pltpu.sync_copy(agg.at[pl.ds(r0, ROWS_PER_TILE)],
                        out_hbm.at[c, pl.ds(r0, ROWS_PER_TILE)])

    return k(x, row, col, zeros_blk)


def _mlp_body(x_ref, p0_ref, p1_ref, w1_ref, b1_ref, w2_ref, b2_ref, o_ref):
    h = x_ref[...] + p0_ref[...] + p1_ref[...]
    h1 = jnp.dot(h, w1_ref[...], preferred_element_type=jnp.float32) + b1_ref[...]
    h1 = jnp.maximum(h1, 0.0)
    o_ref[...] = jnp.dot(h1, w2_ref[...], preferred_element_type=jnp.float32) + b2_ref[...]


def _tc_mlp(x, p0, p1, w1t, b1, w2t, b2):
    block = 2000
    grid = (N_NODES // block,)
    row_spec = pl.BlockSpec((block, D), lambda i: (i, 0))
    full_spec = pl.BlockSpec((D, D), lambda i: (0, 0))
    bias_spec = pl.BlockSpec((1, D), lambda i: (0, 0))
    return pl.pallas_call(
        _mlp_body,
        grid=grid,
        in_specs=[row_spec, row_spec, row_spec, full_spec, bias_spec, full_spec, bias_spec],
        out_specs=row_spec,
        out_shape=jax.ShapeDtypeStruct((N_NODES, D), jnp.float32),
    )(x, p0, p1, w1t, b1, w2t, b2)


@jax.jit
def kernel(x, edge_index, W1, b1, W2, b2):
    pad = E_PAD - N_EDGES
    row = jnp.concatenate(
        [edge_index[0].astype(jnp.int32), jnp.zeros((pad,), jnp.int32)])
    col = jnp.concatenate(
        [edge_index[1].astype(jnp.int32), jnp.full((pad,), N_NODES, jnp.int32)])
    row = row.reshape(NW, N_GROUPS, K_GROUP, EDGE_BATCH)
    col = col.reshape(NW, N_GROUPS, K_GROUP, EDGE_BATCH)
    zeros_blk = jnp.zeros((ROWS_PER_TILE, D), jnp.float32)
    partials = _sc_aggregate(x, row, col, zeros_blk)
    return _tc_mlp(x, partials[0, :N_NODES], partials[1, :N_NODES],
                   W1.T, b1.reshape(1, D), W2.T, b2.reshape(1, D))
